# baseline (speedup 1.0000x reference)
import jax
import jax.numpy as jnp
from jax import lax
from jax.experimental import pallas as pl
from jax.experimental.pallas import tpu as pltpu

B = 2
N = 2048
F = 256
HID = 64
H1 = 4
ACT = 16
TILE = 256
NT = N // TILE
NEG_SLOPE = 0.2
F32 = jnp.float32
BF16 = jnp.bfloat16
VPW = 128
DN_RT = (((1,), (1,)), ((), ()))


def _mask_tile(adj_tile, n0):
    row_ids = n0 + jax.lax.broadcasted_iota(jnp.int32, (TILE, N), 0)
    col_ids = jax.lax.broadcasted_iota(jnp.int32, (TILE, N), 1)
    return (adj_tile > 0) | (row_ids == col_ids)


def _mask_penalty(adj_tile, n0):
    pen = jnp.where(_mask_tile(adj_tile, n0), 0.0, -1e9)
    return pen.astype(BF16)


def _attn_weights(s_col, d_row, pen):
    e = s_col + d_row
    e = jnp.maximum(e, NEG_SLOPE * e)
    e = e + pen
    return jnp.exp(e)


def _body(x_ref, adj_ref,
          w1a_ref, w1ab_ref, asrcp_a_ref, adstT_a_ref,
          w1c_ref, w1cb_ref, asrcp_c_ref, adstT_c_ref,
          act_ref, w2a_ref, a2sap_ref, a2da_ref,
          w2c_ref, a2sc_ref, a2dc_ref,
          lp_ref, val_ref, ent_ref,
          vpa, vpc, dta, dtc, x2a, x2c, vp2a, vp2c, dt2a, dt2c):
    b = pl.program_id(0)
    p = pl.program_id(1)
    i = pl.program_id(2)
    n0 = i * TILE

    @pl.when((b == 0) & (p == 0) & (i == 0))
    def _init_ones():
        for vp in (vpa, vpc):
            vp[...] = jnp.zeros((N, H1 * VPW), BF16)
            for h in range(H1):
                vp[:, h * VPW + HID:h * VPW + HID + 1] = jnp.ones((N, 1), BF16)
        vp2a[...] = jnp.zeros((N, 2 * ACT), BF16)
        vp2a[:, ACT:ACT + 1] = jnp.ones((N, 1), BF16)
        vp2c[...] = jnp.zeros((N, 8), BF16)
        vp2c[:, 1:2] = jnp.ones((N, 1), BF16)

    @pl.when((p == 0) & (i == 0))
    def _l1_prologue():
        x = x_ref[0]
        xb = x.astype(BF16)
        for w_ref, wb_ref, adstT_ref, vp, dt in (
                (w1a_ref, w1ab_ref, adstT_a_ref, vpa, dta),
                (w1c_ref, w1cb_ref, adstT_c_ref, vpc, dtc)):
            hh = jnp.dot(xb, wb_ref[...], preferred_element_type=F32
                         ).astype(BF16)
            for h in range(H1):
                vp[:, h * VPW:h * VPW + HID] = hh[:, h * HID:(h + 1) * HID]
            ma = lax.dot_general(adstT_ref[...], w_ref[...], DN_RT,
                                 preferred_element_type=F32)
            dt[...] = lax.dot_general(ma, x, DN_RT,
                                      preferred_element_type=F32
                                      ).astype(BF16)

    @pl.when(p == 0)
    def _l1_tile():
        pen = _mask_penalty(adj_ref[...], n0)
        for vp, asrcp_ref, dt, x2 in ((vpa, asrcp_a_ref, dta, x2a),
                                      (vpc, asrcp_c_ref, dtc, x2c)):
            rows = vp[pl.ds(n0, TILE), :]
            s = jnp.dot(rows, asrcp_ref[...],
                        preferred_element_type=F32).astype(BF16)
            cols = []
            for h in range(H1):
                w = _attn_weights(s[:, h:h + 1], dt[h:h + 1, :], pen)
                nd = jnp.dot(w, vp[:, h * VPW:(h + 1) * VPW],
                             preferred_element_type=F32)
                cols.append(nd[:, :HID] / nd[:, HID:HID + 1])
            o = jnp.concatenate(cols, axis=1)
            x2[pl.ds(n0, TILE), :] = jnp.where(o > 0, o, jnp.exp(o) - 1.0)

    @pl.when((p == 1) & (i == 0))
    def _l2_prologue():
        xa = x2a[...]
        xc = x2c[...]
        vp2a[:, :ACT] = jnp.dot(xa, w2a_ref[...],
                                preferred_element_type=F32).astype(BF16)
        ma = lax.dot_general(a2da_ref[...], w2a_ref[...], DN_RT,
                             preferred_element_type=F32)
        dt2a[...] = lax.dot_general(ma, xa, DN_RT,
                                    preferred_element_type=F32
                                    ).astype(BF16)
        vp2c[:, 0:1] = jnp.dot(xc, w2c_ref[...],
                               preferred_element_type=F32).astype(BF16)
        mc = lax.dot_general(w2c_ref[...], xc,
                             (((0,), (1,)), ((), ())),
                             preferred_element_type=F32)
        dt2c[...] = (a2dc_ref[0, 0] * mc).astype(BF16)
        lp_ref[...] = jnp.zeros((1, 1, 1), F32)
        ent_ref[...] = jnp.zeros((1, 1, 1), F32)

    @pl.when(p == 1)
    def _l2_tile():
        pen = _mask_penalty(adj_ref[...], n0)

        rows_a = vp2a[pl.ds(n0, TILE), :]
        sa = jnp.dot(rows_a, a2sap_ref[...],
                     preferred_element_type=F32).astype(BF16)
        w = _attn_weights(sa, dt2a[0:1, :], pen)
        nd = jnp.dot(w, vp2a[...], preferred_element_type=F32)
        logits = nd[:, :ACT] / nd[:, ACT:ACT + 1]
        m16 = jnp.max(logits, axis=1, keepdims=True)
        ex = jnp.exp(logits - m16)
        s16 = jnp.sum(ex, axis=1, keepdims=True)
        p_ = ex / s16
        logp = jnp.log(p_ + 1e-12)
        ent = -jnp.sum(p_ * logp, axis=1)
        act = act_ref[0]
        onehot = jax.lax.broadcasted_iota(jnp.int32, (TILE, ACT), 1) == act
        alp = jnp.sum(jnp.where(onehot, logp, 0.0), axis=1)
        lp_ref[...] += jnp.reshape(jnp.sum(alp), (1, 1, 1))
        ent_ref[...] += jnp.reshape(jnp.sum(ent), (1, 1, 1))

        sc = vp2c[pl.ds(n0, TILE), 0:1] * a2sc_ref[0, 0].astype(BF16)
        wc = _attn_weights(sc, dt2c[0:1, :], pen)
        ndc = jnp.dot(wc, vp2c[...], preferred_element_type=F32)
        val_ref[0, pl.ds(n0, TILE), :] = ndc[:, 0:1] / ndc[:, 1:2]


def _pipeline(state, adj, w1a, w1ab, asrcp_a, adstT_a,
              w1c, w1cb, asrcp_c, adstT_c,
              act_col, w2a, a2sap, a2da_row, w2c, a2sc_s, a2dc_s):
    c0 = lambda b, p, i: (0, 0)
    return pl.pallas_call(
        _body,
        grid=(B, 2, NT),
        in_specs=[pl.BlockSpec((1, N, F), lambda b, p, i: (b, 0, 0)),
                  pl.BlockSpec((TILE, N), lambda b, p, i: (i, 0)),
                  pl.BlockSpec((F, F), c0),
                  pl.BlockSpec((F, F), c0),
                  pl.BlockSpec((H1 * VPW, H1), c0),
                  pl.BlockSpec((H1, F), c0),
                  pl.BlockSpec((F, F), c0),
                  pl.BlockSpec((F, F), c0),
                  pl.BlockSpec((H1 * VPW, H1), c0),
                  pl.BlockSpec((H1, F), c0),
                  pl.BlockSpec((1, TILE, 1), lambda b, p, i: (b, i, 0)),
                  pl.BlockSpec((F, ACT), c0),
                  pl.BlockSpec((2 * ACT, 1), c0),
                  pl.BlockSpec((1, ACT), c0),
                  pl.BlockSpec((F, 1), c0),
                  pl.BlockSpec((1, 1), c0),
                  pl.BlockSpec((1, 1), c0)],
        out_specs=[pl.BlockSpec((1, 1, 1), lambda b, p, i: (b, 0, 0)),
                   pl.BlockSpec((1, N, 1), lambda b, p, i: (b, 0, 0)),
                   pl.BlockSpec((1, 1, 1), lambda b, p, i: (b, 0, 0))],
        out_shape=[jax.ShapeDtypeStruct((B, 1, 1), F32),
                   jax.ShapeDtypeStruct((B, N, 1), F32),
                   jax.ShapeDtypeStruct((B, 1, 1), F32)],
        scratch_shapes=[pltpu.VMEM((N, H1 * VPW), BF16),
                        pltpu.VMEM((N, H1 * VPW), BF16),
                        pltpu.VMEM((H1, N), BF16), pltpu.VMEM((H1, N), BF16),
                        pltpu.VMEM((N, F), F32), pltpu.VMEM((N, F), F32),
                        pltpu.VMEM((N, 2 * ACT), BF16), pltpu.VMEM((N, 8), BF16),
                        pltpu.VMEM((1, N), BF16), pltpu.VMEM((1, N), BF16)],
    )(state, adj, w1a, w1ab, asrcp_a, adstT_a, w1c, w1cb, asrcp_c,
      adstT_c, act_col, w2a, a2sap, a2da_row, w2c, a2sc_s, a2dc_s)


def kernel(state, adj, action, W1a, a1sa, a1da, W2a, a2sa, a2da,
           W1c, a1sc, a1dc, W2c, a2sc, a2dc):
    state = state.astype(F32)
    eye = jnp.eye(H1, dtype=F32)

    def l1_prep(W1, a1s, a1d):
        w1 = W1.reshape(F, H1 * HID)
        asrc = a1s[:, :, None] * eye[:, None, :]
        asrcp = jnp.pad(asrc, ((0, 0), (0, VPW - HID), (0, 0))
                        ).reshape(H1 * VPW, H1)
        adstT = (eye[:, :, None] * a1d[None, :, :]).reshape(H1, H1 * HID)
        return w1, asrcp.astype(BF16), adstT

    w1a, asrcp_a, adstT_a = l1_prep(W1a, a1sa, a1da)
    w1c, asrcp_c, adstT_c = l1_prep(W1c, a1sc, a1dc)

    act_col = action.astype(jnp.int32).reshape(B, N, 1)
    w2a = W2a.reshape(F, ACT)
    w2c = W2c.reshape(F, 1)
    a2sap = jnp.pad(a2sa.reshape(ACT, 1), ((0, ACT), (0, 0))).astype(BF16)

    lp, val, ent = _pipeline(
        state, adj, w1a, w1a.astype(BF16), asrcp_a, adstT_a,
        w1c, w1c.astype(BF16), asrcp_c, adstT_c,
        act_col, w2a, a2sap, a2da.reshape(1, ACT),
        w2c, a2sc.reshape(1, 1), a2dc.reshape(1, 1))

    return (lp.reshape(B), val.reshape(B, N), ent.reshape(B))


# speedup vs baseline: 1.0227x; 1.0227x over previous
import jax
import jax.numpy as jnp
from jax import lax
from jax.experimental import pallas as pl
from jax.experimental.pallas import tpu as pltpu

B = 2
N = 2048
F = 256
HID = 64
H1 = 4
ACT = 16
TILE = 256
NT = N // TILE
NEG_SLOPE = 0.2
F32 = jnp.float32
BF16 = jnp.bfloat16
VPW = 128
DN_RT = (((1,), (1,)), ((), ()))


def _mask_tile(adj_tile, n0):
    row_ids = n0 + jax.lax.broadcasted_iota(jnp.int32, (TILE, N), 0)
    col_ids = jax.lax.broadcasted_iota(jnp.int32, (TILE, N), 1)
    return (adj_tile > 0) | (row_ids == col_ids)


def _mask_penalty(adj_tile, n0):
    pen = jnp.where(_mask_tile(adj_tile, n0), 0.0, -1e9)
    return pen.astype(BF16)


def _attn_weights(s_col, d_row, pen):
    e = s_col + d_row
    e = jnp.maximum(e, NEG_SLOPE * e)
    e = e + pen
    return jnp.exp(e)


def _body(x_ref, adj_ref,
          w1a_ref, w1ab_ref, asrcp_a_ref, adstT_a_ref,
          w1c_ref, w1cb_ref, asrcp_c_ref, adstT_c_ref,
          act_ref, w2a_ref, a2sap_ref, a2da_ref,
          w2c_ref, a2sc_ref, a2dc_ref,
          lp_ref, val_ref, ent_ref,
          vpa, vpc, dta, dtc, x2a, x2c, vp2a, vp2c, dt2a, dt2c, penc):
    b = pl.program_id(0)
    p = pl.program_id(1)
    i = pl.program_id(2)
    n0 = i * TILE

    @pl.when((b == 0) & (p == 0) & (i == 0))
    def _init_ones():
        for vp in (vpa, vpc):
            vp[...] = jnp.zeros((N, H1 * VPW), BF16)
            for h in range(H1):
                vp[:, h * VPW + HID:h * VPW + HID + 1] = jnp.ones((N, 1), BF16)
        vp2a[...] = jnp.zeros((N, 2 * ACT), BF16)
        vp2a[:, ACT:ACT + 1] = jnp.ones((N, 1), BF16)
        vp2c[...] = jnp.zeros((N, 8), BF16)
        vp2c[:, 1:2] = jnp.ones((N, 1), BF16)

    @pl.when((p == 0) & (i == 0))
    def _l1_prologue():
        x = x_ref[0]
        xb = x.astype(BF16)
        for w_ref, wb_ref, adstT_ref, vp, dt in (
                (w1a_ref, w1ab_ref, adstT_a_ref, vpa, dta),
                (w1c_ref, w1cb_ref, adstT_c_ref, vpc, dtc)):
            hh = jnp.dot(xb, wb_ref[...], preferred_element_type=F32
                         ).astype(BF16)
            for h in range(H1):
                vp[:, h * VPW:h * VPW + HID] = hh[:, h * HID:(h + 1) * HID]
            ma = lax.dot_general(adstT_ref[...], w_ref[...], DN_RT,
                                 preferred_element_type=F32)
            dt[...] = lax.dot_general(ma, x, DN_RT,
                                      preferred_element_type=F32
                                      ).astype(BF16)

    @pl.when((b == 0) & (p == 0))
    def _pen_tile():
        penc[pl.ds(n0, TILE), :] = _mask_penalty(adj_ref[...], n0)

    @pl.when(p == 0)
    def _l1_tile():
        pen = penc[pl.ds(n0, TILE), :]
        for vp, asrcp_ref, dt, x2 in ((vpa, asrcp_a_ref, dta, x2a),
                                      (vpc, asrcp_c_ref, dtc, x2c)):
            rows = vp[pl.ds(n0, TILE), :]
            s = jnp.dot(rows, asrcp_ref[...],
                        preferred_element_type=F32).astype(BF16)
            cols = []
            for h in range(H1):
                w = _attn_weights(s[:, h:h + 1], dt[h:h + 1, :], pen)
                nd = jnp.dot(w, vp[:, h * VPW:(h + 1) * VPW],
                             preferred_element_type=F32)
                cols.append(nd[:, :HID] / nd[:, HID:HID + 1])
            o = jnp.concatenate(cols, axis=1)
            x2[pl.ds(n0, TILE), :] = jnp.where(o > 0, o, jnp.exp(o) - 1.0)

    @pl.when((p == 1) & (i == 0))
    def _l2_prologue():
        xa = x2a[...]
        xc = x2c[...]
        vp2a[:, :ACT] = jnp.dot(xa, w2a_ref[...],
                                preferred_element_type=F32).astype(BF16)
        ma = lax.dot_general(a2da_ref[...], w2a_ref[...], DN_RT,
                             preferred_element_type=F32)
        dt2a[...] = lax.dot_general(ma, xa, DN_RT,
                                    preferred_element_type=F32
                                    ).astype(BF16)
        vp2c[:, 0:1] = jnp.dot(xc, w2c_ref[...],
                               preferred_element_type=F32).astype(BF16)
        mc = lax.dot_general(w2c_ref[...], xc,
                             (((0,), (1,)), ((), ())),
                             preferred_element_type=F32)
        dt2c[...] = (a2dc_ref[0, 0] * mc).astype(BF16)
        lp_ref[...] = jnp.zeros((1, 1, 1), F32)
        ent_ref[...] = jnp.zeros((1, 1, 1), F32)

    @pl.when(p == 1)
    def _l2_tile():
        pen = penc[pl.ds(n0, TILE), :]

        rows_a = vp2a[pl.ds(n0, TILE), :]
        sa = jnp.dot(rows_a, a2sap_ref[...],
                     preferred_element_type=F32).astype(BF16)
        w = _attn_weights(sa, dt2a[0:1, :], pen)
        nd = jnp.dot(w, vp2a[...], preferred_element_type=F32)
        logits = nd[:, :ACT] / nd[:, ACT:ACT + 1]
        m16 = jnp.max(logits, axis=1, keepdims=True)
        ex = jnp.exp(logits - m16)
        s16 = jnp.sum(ex, axis=1, keepdims=True)
        p_ = ex / s16
        logp = jnp.log(p_ + 1e-12)
        ent = -jnp.sum(p_ * logp, axis=1)
        act = act_ref[0]
        onehot = jax.lax.broadcasted_iota(jnp.int32, (TILE, ACT), 1) == act
        alp = jnp.sum(jnp.where(onehot, logp, 0.0), axis=1)
        lp_ref[...] += jnp.reshape(jnp.sum(alp), (1, 1, 1))
        ent_ref[...] += jnp.reshape(jnp.sum(ent), (1, 1, 1))

        sc = vp2c[pl.ds(n0, TILE), 0:1] * a2sc_ref[0, 0].astype(BF16)
        wc = _attn_weights(sc, dt2c[0:1, :], pen)
        ndc = jnp.dot(wc, vp2c[...], preferred_element_type=F32)
        val_ref[0, pl.ds(n0, TILE), :] = ndc[:, 0:1] / ndc[:, 1:2]


def _pipeline(state, adj, w1a, w1ab, asrcp_a, adstT_a,
              w1c, w1cb, asrcp_c, adstT_c,
              act_col, w2a, a2sap, a2da_row, w2c, a2sc_s, a2dc_s):
    c0 = lambda b, p, i: (0, 0)
    return pl.pallas_call(
        _body,
        grid=(B, 2, NT),
        in_specs=[pl.BlockSpec((1, N, F), lambda b, p, i: (b, 0, 0)),
                  pl.BlockSpec((TILE, N), lambda b, p, i: (i, 0)),
                  pl.BlockSpec((F, F), c0),
                  pl.BlockSpec((F, F), c0),
                  pl.BlockSpec((H1 * VPW, H1), c0),
                  pl.BlockSpec((H1, F), c0),
                  pl.BlockSpec((F, F), c0),
                  pl.BlockSpec((F, F), c0),
                  pl.BlockSpec((H1 * VPW, H1), c0),
                  pl.BlockSpec((H1, F), c0),
                  pl.BlockSpec((1, TILE, 1), lambda b, p, i: (b, i, 0)),
                  pl.BlockSpec((F, ACT), c0),
                  pl.BlockSpec((2 * ACT, 1), c0),
                  pl.BlockSpec((1, ACT), c0),
                  pl.BlockSpec((F, 1), c0),
                  pl.BlockSpec((1, 1), c0),
                  pl.BlockSpec((1, 1), c0)],
        out_specs=[pl.BlockSpec((1, 1, 1), lambda b, p, i: (b, 0, 0)),
                   pl.BlockSpec((1, N, 1), lambda b, p, i: (b, 0, 0)),
                   pl.BlockSpec((1, 1, 1), lambda b, p, i: (b, 0, 0))],
        out_shape=[jax.ShapeDtypeStruct((B, 1, 1), F32),
                   jax.ShapeDtypeStruct((B, N, 1), F32),
                   jax.ShapeDtypeStruct((B, 1, 1), F32)],
        scratch_shapes=[pltpu.VMEM((N, H1 * VPW), BF16),
                        pltpu.VMEM((N, H1 * VPW), BF16),
                        pltpu.VMEM((H1, N), BF16), pltpu.VMEM((H1, N), BF16),
                        pltpu.VMEM((N, F), F32), pltpu.VMEM((N, F), F32),
                        pltpu.VMEM((N, 2 * ACT), BF16), pltpu.VMEM((N, 8), BF16),
                        pltpu.VMEM((1, N), BF16), pltpu.VMEM((1, N), BF16),
                        pltpu.VMEM((N, N), BF16)],
    )(state, adj, w1a, w1ab, asrcp_a, adstT_a, w1c, w1cb, asrcp_c,
      adstT_c, act_col, w2a, a2sap, a2da_row, w2c, a2sc_s, a2dc_s)


def kernel(state, adj, action, W1a, a1sa, a1da, W2a, a2sa, a2da,
           W1c, a1sc, a1dc, W2c, a2sc, a2dc):
    state = state.astype(F32)
    eye = jnp.eye(H1, dtype=F32)

    def l1_prep(W1, a1s, a1d):
        w1 = W1.reshape(F, H1 * HID)
        asrc = a1s[:, :, None] * eye[:, None, :]
        asrcp = jnp.pad(asrc, ((0, 0), (0, VPW - HID), (0, 0))
                        ).reshape(H1 * VPW, H1)
        adstT = (eye[:, :, None] * a1d[None, :, :]).reshape(H1, H1 * HID)
        return w1, asrcp.astype(BF16), adstT

    w1a, asrcp_a, adstT_a = l1_prep(W1a, a1sa, a1da)
    w1c, asrcp_c, adstT_c = l1_prep(W1c, a1sc, a1dc)

    act_col = action.astype(jnp.int32).reshape(B, N, 1)
    w2a = W2a.reshape(F, ACT)
    w2c = W2c.reshape(F, 1)
    a2sap = jnp.pad(a2sa.reshape(ACT, 1), ((0, ACT), (0, 0))).astype(BF16)

    lp, val, ent = _pipeline(
        state, adj, w1a, w1a.astype(BF16), asrcp_a, adstT_a,
        w1c, w1c.astype(BF16), asrcp_c, adstT_c,
        act_col, w2a, a2sap, a2da.reshape(1, ACT),
        w2c, a2sc.reshape(1, 1), a2dc.reshape(1, 1))

    return (lp.reshape(B), val.reshape(B, N), ent.reshape(B))


# speedup vs baseline: 1.1325x; 1.1074x over previous
import jax
import jax.numpy as jnp
from jax import lax
from jax.experimental import pallas as pl
from jax.experimental.pallas import tpu as pltpu

B = 2
N = 2048
F = 256
HID = 64
H1 = 4
ACT = 16
TILE = 512
NT = N // TILE
NEG_SLOPE = 0.2
F32 = jnp.float32
BF16 = jnp.bfloat16
VPW = 128
DN_RT = (((1,), (1,)), ((), ()))


def _mask_tile(adj_tile, n0):
    row_ids = n0 + jax.lax.broadcasted_iota(jnp.int32, (TILE, N), 0)
    col_ids = jax.lax.broadcasted_iota(jnp.int32, (TILE, N), 1)
    return (adj_tile > 0) | (row_ids == col_ids)


def _mask_penalty(adj_tile, n0):
    pen = jnp.where(_mask_tile(adj_tile, n0), 0.0, -1e9)
    return pen.astype(BF16)


def _attn_weights(s_col, d_row, pen):
    e = s_col + d_row
    e = jnp.maximum(e, NEG_SLOPE * e)
    e = e + pen
    return jnp.exp(e)


def _body(x_ref, adj_ref,
          w1a_ref, w1ab_ref, asrcp_a_ref, adstT_a_ref,
          w1c_ref, w1cb_ref, asrcp_c_ref, adstT_c_ref,
          act_ref, w2a_ref, a2sap_ref, a2da_ref,
          w2c_ref, a2sc_ref, a2dc_ref,
          lp_ref, val_ref, ent_ref,
          vpa, vpc, dta, dtc, x2a, x2c, vp2a, vp2c, dt2a, dt2c, penc):
    b = pl.program_id(0)
    p = pl.program_id(1)
    i = pl.program_id(2)
    n0 = i * TILE

    @pl.when((b == 0) & (p == 0) & (i == 0))
    def _init_ones():
        for vp in (vpa, vpc):
            vp[...] = jnp.zeros((N, H1 * VPW), BF16)
            for h in range(H1):
                vp[:, h * VPW + HID:h * VPW + HID + 1] = jnp.ones((N, 1), BF16)
        vp2a[...] = jnp.zeros((N, 2 * ACT), BF16)
        vp2a[:, ACT:ACT + 1] = jnp.ones((N, 1), BF16)
        vp2c[...] = jnp.zeros((N, 8), BF16)
        vp2c[:, 1:2] = jnp.ones((N, 1), BF16)

    @pl.when((p == 0) & (i == 0))
    def _l1_prologue():
        x = x_ref[0]
        xb = x.astype(BF16)
        for w_ref, wb_ref, adstT_ref, vp, dt in (
                (w1a_ref, w1ab_ref, adstT_a_ref, vpa, dta),
                (w1c_ref, w1cb_ref, adstT_c_ref, vpc, dtc)):
            hh = jnp.dot(xb, wb_ref[...], preferred_element_type=F32
                         ).astype(BF16)
            for h in range(H1):
                vp[:, h * VPW:h * VPW + HID] = hh[:, h * HID:(h + 1) * HID]
            ma = lax.dot_general(adstT_ref[...], w_ref[...], DN_RT,
                                 preferred_element_type=F32)
            dt[...] = lax.dot_general(ma, x, DN_RT,
                                      preferred_element_type=F32
                                      ).astype(BF16)

    @pl.when((b == 0) & (p == 0))
    def _pen_tile():
        penc[pl.ds(n0, TILE), :] = _mask_penalty(adj_ref[...], n0)

    @pl.when(p == 0)
    def _l1_tile():
        pen = penc[pl.ds(n0, TILE), :]
        for vp, asrcp_ref, dt, x2 in ((vpa, asrcp_a_ref, dta, x2a),
                                      (vpc, asrcp_c_ref, dtc, x2c)):
            rows = vp[pl.ds(n0, TILE), :]
            s = jnp.dot(rows, asrcp_ref[...],
                        preferred_element_type=F32).astype(BF16)
            cols = []
            for h in range(H1):
                w = _attn_weights(s[:, h:h + 1], dt[h:h + 1, :], pen)
                nd = jnp.dot(w, vp[:, h * VPW:(h + 1) * VPW],
                             preferred_element_type=F32)
                cols.append(nd[:, :HID] / nd[:, HID:HID + 1])
            o = jnp.concatenate(cols, axis=1)
            x2[pl.ds(n0, TILE), :] = jnp.where(o > 0, o, jnp.exp(o) - 1.0)

    @pl.when((p == 1) & (i == 0))
    def _l2_prologue():
        xa = x2a[...]
        xc = x2c[...]
        vp2a[:, :ACT] = jnp.dot(xa, w2a_ref[...],
                                preferred_element_type=F32).astype(BF16)
        ma = lax.dot_general(a2da_ref[...], w2a_ref[...], DN_RT,
                             preferred_element_type=F32)
        dt2a[...] = lax.dot_general(ma, xa, DN_RT,
                                    preferred_element_type=F32
                                    ).astype(BF16)
        vp2c[:, 0:1] = jnp.dot(xc, w2c_ref[...],
                               preferred_element_type=F32).astype(BF16)
        mc = lax.dot_general(w2c_ref[...], xc,
                             (((0,), (1,)), ((), ())),
                             preferred_element_type=F32)
        dt2c[...] = (a2dc_ref[0, 0] * mc).astype(BF16)
        lp_ref[...] = jnp.zeros((1, 1, 1), F32)
        ent_ref[...] = jnp.zeros((1, 1, 1), F32)

    @pl.when(p == 1)
    def _l2_tile():
        pen = penc[pl.ds(n0, TILE), :]

        rows_a = vp2a[pl.ds(n0, TILE), :]
        sa = jnp.dot(rows_a, a2sap_ref[...],
                     preferred_element_type=F32).astype(BF16)
        w = _attn_weights(sa, dt2a[0:1, :], pen)
        nd = jnp.dot(w, vp2a[...], preferred_element_type=F32)
        logits = nd[:, :ACT] / nd[:, ACT:ACT + 1]
        m16 = jnp.max(logits, axis=1, keepdims=True)
        ex = jnp.exp(logits - m16)
        s16 = jnp.sum(ex, axis=1, keepdims=True)
        p_ = ex / s16
        logp = jnp.log(p_ + 1e-12)
        ent = -jnp.sum(p_ * logp, axis=1)
        act = act_ref[0]
        onehot = jax.lax.broadcasted_iota(jnp.int32, (TILE, ACT), 1) == act
        alp = jnp.sum(jnp.where(onehot, logp, 0.0), axis=1)
        lp_ref[...] += jnp.reshape(jnp.sum(alp), (1, 1, 1))
        ent_ref[...] += jnp.reshape(jnp.sum(ent), (1, 1, 1))

        sc = vp2c[pl.ds(n0, TILE), 0:1] * a2sc_ref[0, 0].astype(BF16)
        wc = _attn_weights(sc, dt2c[0:1, :], pen)
        ndc = jnp.dot(wc, vp2c[...], preferred_element_type=F32)
        val_ref[0, pl.ds(n0, TILE), :] = ndc[:, 0:1] / ndc[:, 1:2]


def _pipeline(state, adj, w1a, w1ab, asrcp_a, adstT_a,
              w1c, w1cb, asrcp_c, adstT_c,
              act_col, w2a, a2sap, a2da_row, w2c, a2sc_s, a2dc_s):
    c0 = lambda b, p, i: (0, 0)
    return pl.pallas_call(
        _body,
        grid=(B, 2, NT),
        in_specs=[pl.BlockSpec((1, N, F), lambda b, p, i: (b, 0, 0)),
                  pl.BlockSpec((TILE, N), lambda b, p, i: (i, 0)),
                  pl.BlockSpec((F, F), c0),
                  pl.BlockSpec((F, F), c0),
                  pl.BlockSpec((H1 * VPW, H1), c0),
                  pl.BlockSpec((H1, F), c0),
                  pl.BlockSpec((F, F), c0),
                  pl.BlockSpec((F, F), c0),
                  pl.BlockSpec((H1 * VPW, H1), c0),
                  pl.BlockSpec((H1, F), c0),
                  pl.BlockSpec((1, TILE, 1), lambda b, p, i: (b, i, 0)),
                  pl.BlockSpec((F, ACT), c0),
                  pl.BlockSpec((2 * ACT, 1), c0),
                  pl.BlockSpec((1, ACT), c0),
                  pl.BlockSpec((F, 1), c0),
                  pl.BlockSpec((1, 1), c0),
                  pl.BlockSpec((1, 1), c0)],
        out_specs=[pl.BlockSpec((1, 1, 1), lambda b, p, i: (b, 0, 0)),
                   pl.BlockSpec((1, N, 1), lambda b, p, i: (b, 0, 0)),
                   pl.BlockSpec((1, 1, 1), lambda b, p, i: (b, 0, 0))],
        out_shape=[jax.ShapeDtypeStruct((B, 1, 1), F32),
                   jax.ShapeDtypeStruct((B, N, 1), F32),
                   jax.ShapeDtypeStruct((B, 1, 1), F32)],
        scratch_shapes=[pltpu.VMEM((N, H1 * VPW), BF16),
                        pltpu.VMEM((N, H1 * VPW), BF16),
                        pltpu.VMEM((H1, N), BF16), pltpu.VMEM((H1, N), BF16),
                        pltpu.VMEM((N, F), F32), pltpu.VMEM((N, F), F32),
                        pltpu.VMEM((N, 2 * ACT), BF16), pltpu.VMEM((N, 8), BF16),
                        pltpu.VMEM((1, N), BF16), pltpu.VMEM((1, N), BF16),
                        pltpu.VMEM((N, N), BF16)],
    )(state, adj, w1a, w1ab, asrcp_a, adstT_a, w1c, w1cb, asrcp_c,
      adstT_c, act_col, w2a, a2sap, a2da_row, w2c, a2sc_s, a2dc_s)


def kernel(state, adj, action, W1a, a1sa, a1da, W2a, a2sa, a2da,
           W1c, a1sc, a1dc, W2c, a2sc, a2dc):
    state = state.astype(F32)
    eye = jnp.eye(H1, dtype=F32)

    def l1_prep(W1, a1s, a1d):
        w1 = W1.reshape(F, H1 * HID)
        asrc = a1s[:, :, None] * eye[:, None, :]
        asrcp = jnp.pad(asrc, ((0, 0), (0, VPW - HID), (0, 0))
                        ).reshape(H1 * VPW, H1)
        adstT = (eye[:, :, None] * a1d[None, :, :]).reshape(H1, H1 * HID)
        return w1, asrcp.astype(BF16), adstT

    w1a, asrcp_a, adstT_a = l1_prep(W1a, a1sa, a1da)
    w1c, asrcp_c, adstT_c = l1_prep(W1c, a1sc, a1dc)

    act_col = action.astype(jnp.int32).reshape(B, N, 1)
    w2a = W2a.reshape(F, ACT)
    w2c = W2c.reshape(F, 1)
    a2sap = jnp.pad(a2sa.reshape(ACT, 1), ((0, ACT), (0, 0))).astype(BF16)

    lp, val, ent = _pipeline(
        state, adj, w1a, w1a.astype(BF16), asrcp_a, adstT_a,
        w1c, w1c.astype(BF16), asrcp_c, adstT_c,
        act_col, w2a, a2sap, a2da.reshape(1, ACT),
        w2c, a2sc.reshape(1, 1), a2dc.reshape(1, 1))

    return (lp.reshape(B), val.reshape(B, N), ent.reshape(B))


# speedup vs baseline: 1.1645x; 1.0283x over previous
import jax
import jax.numpy as jnp
from jax import lax
from jax.experimental import pallas as pl
from jax.experimental.pallas import tpu as pltpu

B = 2
N = 2048
F = 256
HID = 64
H1 = 4
ACT = 16
TILE = 1024
NT = N // TILE
NEG_SLOPE = 0.2
F32 = jnp.float32
BF16 = jnp.bfloat16
VPW = 128
DN_RT = (((1,), (1,)), ((), ()))


def _mask_tile(adj_tile, n0):
    row_ids = n0 + jax.lax.broadcasted_iota(jnp.int32, (TILE, N), 0)
    col_ids = jax.lax.broadcasted_iota(jnp.int32, (TILE, N), 1)
    return (adj_tile > 0) | (row_ids == col_ids)


def _mask_penalty(adj_tile, n0):
    pen = jnp.where(_mask_tile(adj_tile, n0), 0.0, -1e9)
    return pen.astype(BF16)


def _attn_weights(s_col, d_row, pen):
    e = s_col + d_row
    e = jnp.maximum(e, NEG_SLOPE * e)
    e = e + pen
    return jnp.exp(e)


def _body(x_ref, adj_ref,
          w1a_ref, w1ab_ref, asrcp_a_ref, adstT_a_ref,
          w1c_ref, w1cb_ref, asrcp_c_ref, adstT_c_ref,
          act_ref, w2a_ref, a2sap_ref, a2da_ref,
          w2c_ref, a2sc_ref, a2dc_ref,
          lp_ref, val_ref, ent_ref,
          vpa, vpc, dta, dtc, x2a, x2c, vp2a, vp2c, dt2a, dt2c, penc):
    b = pl.program_id(0)
    p = pl.program_id(1)
    i = pl.program_id(2)
    n0 = i * TILE

    @pl.when((b == 0) & (p == 0) & (i == 0))
    def _init_ones():
        for vp in (vpa, vpc):
            vp[...] = jnp.zeros((N, H1 * VPW), BF16)
            for h in range(H1):
                vp[:, h * VPW + HID:h * VPW + HID + 1] = jnp.ones((N, 1), BF16)
        vp2a[...] = jnp.zeros((N, 2 * ACT), BF16)
        vp2a[:, ACT:ACT + 1] = jnp.ones((N, 1), BF16)
        vp2c[...] = jnp.zeros((N, 8), BF16)
        vp2c[:, 1:2] = jnp.ones((N, 1), BF16)

    @pl.when((p == 0) & (i == 0))
    def _l1_prologue():
        x = x_ref[0]
        xb = x.astype(BF16)
        for w_ref, wb_ref, adstT_ref, vp, dt in (
                (w1a_ref, w1ab_ref, adstT_a_ref, vpa, dta),
                (w1c_ref, w1cb_ref, adstT_c_ref, vpc, dtc)):
            hh = jnp.dot(xb, wb_ref[...], preferred_element_type=F32
                         ).astype(BF16)
            for h in range(H1):
                vp[:, h * VPW:h * VPW + HID] = hh[:, h * HID:(h + 1) * HID]
            ma = lax.dot_general(adstT_ref[...], w_ref[...], DN_RT,
                                 preferred_element_type=F32)
            dt[...] = lax.dot_general(ma, x, DN_RT,
                                      preferred_element_type=F32
                                      ).astype(BF16)

    @pl.when((b == 0) & (p == 0))
    def _pen_tile():
        penc[pl.ds(n0, TILE), :] = _mask_penalty(adj_ref[...], n0)

    @pl.when(p == 0)
    def _l1_tile():
        pen = penc[pl.ds(n0, TILE), :]
        for vp, asrcp_ref, dt, x2 in ((vpa, asrcp_a_ref, dta, x2a),
                                      (vpc, asrcp_c_ref, dtc, x2c)):
            rows = vp[pl.ds(n0, TILE), :]
            s = jnp.dot(rows, asrcp_ref[...],
                        preferred_element_type=F32).astype(BF16)
            cols = []
            for h in range(H1):
                w = _attn_weights(s[:, h:h + 1], dt[h:h + 1, :], pen)
                nd = jnp.dot(w, vp[:, h * VPW:(h + 1) * VPW],
                             preferred_element_type=F32)
                cols.append(nd[:, :HID] / nd[:, HID:HID + 1])
            o = jnp.concatenate(cols, axis=1)
            x2[pl.ds(n0, TILE), :] = jnp.where(o > 0, o, jnp.exp(o) - 1.0)

    @pl.when((p == 1) & (i == 0))
    def _l2_prologue():
        xa = x2a[...]
        xc = x2c[...]
        vp2a[:, :ACT] = jnp.dot(xa, w2a_ref[...],
                                preferred_element_type=F32).astype(BF16)
        ma = lax.dot_general(a2da_ref[...], w2a_ref[...], DN_RT,
                             preferred_element_type=F32)
        dt2a[...] = lax.dot_general(ma, xa, DN_RT,
                                    preferred_element_type=F32
                                    ).astype(BF16)
        vp2c[:, 0:1] = jnp.dot(xc, w2c_ref[...],
                               preferred_element_type=F32).astype(BF16)
        mc = lax.dot_general(w2c_ref[...], xc,
                             (((0,), (1,)), ((), ())),
                             preferred_element_type=F32)
        dt2c[...] = (a2dc_ref[0, 0] * mc).astype(BF16)
        lp_ref[...] = jnp.zeros((1, 1, 1), F32)
        ent_ref[...] = jnp.zeros((1, 1, 1), F32)

    @pl.when(p == 1)
    def _l2_tile():
        pen = penc[pl.ds(n0, TILE), :]

        rows_a = vp2a[pl.ds(n0, TILE), :]
        sa = jnp.dot(rows_a, a2sap_ref[...],
                     preferred_element_type=F32).astype(BF16)
        w = _attn_weights(sa, dt2a[0:1, :], pen)
        nd = jnp.dot(w, vp2a[...], preferred_element_type=F32)
        logits = nd[:, :ACT] / nd[:, ACT:ACT + 1]
        m16 = jnp.max(logits, axis=1, keepdims=True)
        ex = jnp.exp(logits - m16)
        s16 = jnp.sum(ex, axis=1, keepdims=True)
        p_ = ex / s16
        logp = jnp.log(p_ + 1e-12)
        ent = -jnp.sum(p_ * logp, axis=1)
        act = act_ref[0]
        onehot = jax.lax.broadcasted_iota(jnp.int32, (TILE, ACT), 1) == act
        alp = jnp.sum(jnp.where(onehot, logp, 0.0), axis=1)
        lp_ref[...] += jnp.reshape(jnp.sum(alp), (1, 1, 1))
        ent_ref[...] += jnp.reshape(jnp.sum(ent), (1, 1, 1))

        sc = vp2c[pl.ds(n0, TILE), 0:1] * a2sc_ref[0, 0].astype(BF16)
        wc = _attn_weights(sc, dt2c[0:1, :], pen)
        ndc = jnp.dot(wc, vp2c[...], preferred_element_type=F32)
        val_ref[0, pl.ds(n0, TILE), :] = ndc[:, 0:1] / ndc[:, 1:2]


def _pipeline(state, adj, w1a, w1ab, asrcp_a, adstT_a,
              w1c, w1cb, asrcp_c, adstT_c,
              act_col, w2a, a2sap, a2da_row, w2c, a2sc_s, a2dc_s):
    c0 = lambda b, p, i: (0, 0)
    return pl.pallas_call(
        _body,
        grid=(B, 2, NT),
        in_specs=[pl.BlockSpec((1, N, F), lambda b, p, i: (b, 0, 0)),
                  pl.BlockSpec((TILE, N), lambda b, p, i: (i, 0)),
                  pl.BlockSpec((F, F), c0),
                  pl.BlockSpec((F, F), c0),
                  pl.BlockSpec((H1 * VPW, H1), c0),
                  pl.BlockSpec((H1, F), c0),
                  pl.BlockSpec((F, F), c0),
                  pl.BlockSpec((F, F), c0),
                  pl.BlockSpec((H1 * VPW, H1), c0),
                  pl.BlockSpec((H1, F), c0),
                  pl.BlockSpec((1, TILE, 1), lambda b, p, i: (b, i, 0)),
                  pl.BlockSpec((F, ACT), c0),
                  pl.BlockSpec((2 * ACT, 1), c0),
                  pl.BlockSpec((1, ACT), c0),
                  pl.BlockSpec((F, 1), c0),
                  pl.BlockSpec((1, 1), c0),
                  pl.BlockSpec((1, 1), c0)],
        out_specs=[pl.BlockSpec((1, 1, 1), lambda b, p, i: (b, 0, 0)),
                   pl.BlockSpec((1, N, 1), lambda b, p, i: (b, 0, 0)),
                   pl.BlockSpec((1, 1, 1), lambda b, p, i: (b, 0, 0))],
        out_shape=[jax.ShapeDtypeStruct((B, 1, 1), F32),
                   jax.ShapeDtypeStruct((B, N, 1), F32),
                   jax.ShapeDtypeStruct((B, 1, 1), F32)],
        scratch_shapes=[pltpu.VMEM((N, H1 * VPW), BF16),
                        pltpu.VMEM((N, H1 * VPW), BF16),
                        pltpu.VMEM((H1, N), BF16), pltpu.VMEM((H1, N), BF16),
                        pltpu.VMEM((N, F), F32), pltpu.VMEM((N, F), F32),
                        pltpu.VMEM((N, 2 * ACT), BF16), pltpu.VMEM((N, 8), BF16),
                        pltpu.VMEM((1, N), BF16), pltpu.VMEM((1, N), BF16),
                        pltpu.VMEM((N, N), BF16)],
    )(state, adj, w1a, w1ab, asrcp_a, adstT_a, w1c, w1cb, asrcp_c,
      adstT_c, act_col, w2a, a2sap, a2da_row, w2c, a2sc_s, a2dc_s)


def kernel(state, adj, action, W1a, a1sa, a1da, W2a, a2sa, a2da,
           W1c, a1sc, a1dc, W2c, a2sc, a2dc):
    state = state.astype(F32)
    eye = jnp.eye(H1, dtype=F32)

    def l1_prep(W1, a1s, a1d):
        w1 = W1.reshape(F, H1 * HID)
        asrc = a1s[:, :, None] * eye[:, None, :]
        asrcp = jnp.pad(asrc, ((0, 0), (0, VPW - HID), (0, 0))
                        ).reshape(H1 * VPW, H1)
        adstT = (eye[:, :, None] * a1d[None, :, :]).reshape(H1, H1 * HID)
        return w1, asrcp.astype(BF16), adstT

    w1a, asrcp_a, adstT_a = l1_prep(W1a, a1sa, a1da)
    w1c, asrcp_c, adstT_c = l1_prep(W1c, a1sc, a1dc)

    act_col = action.astype(jnp.int32).reshape(B, N, 1)
    w2a = W2a.reshape(F, ACT)
    w2c = W2c.reshape(F, 1)
    a2sap = jnp.pad(a2sa.reshape(ACT, 1), ((0, ACT), (0, 0))).astype(BF16)

    lp, val, ent = _pipeline(
        state, adj, w1a, w1a.astype(BF16), asrcp_a, adstT_a,
        w1c, w1c.astype(BF16), asrcp_c, adstT_c,
        act_col, w2a, a2sap, a2da.reshape(1, ACT),
        w2c, a2sc.reshape(1, 1), a2dc.reshape(1, 1))

    return (lp.reshape(B), val.reshape(B, N), ent.reshape(B))
